# baseline (speedup 1.0000x reference)
import functools

import jax
import jax.numpy as jnp
from jax import lax
from jax.experimental import pallas as pl
from jax.experimental.pallas import tpu as pltpu
from jax.experimental.pallas import tpu_sc as plsc

BATCH = 16384
EMBED = 64
LANES = 16

_NS = 16
_NW = _NS
_B_PER_W = BATCH // _NW
_PASS = 512
_NPASS = _B_PER_W // _PASS
_CHUNK = 128
_NCHUNK = _PASS // _CHUNK

_mesh = plsc.VectorSubcoreMesh(core_axis_name="c", subcore_axis_name="s", num_cores=1)


def _wid():
    return lax.axis_index("s")


@functools.partial(
    pl.kernel,
    mesh=_mesh,
    out_type=jax.ShapeDtypeStruct((BATCH, EMBED), jnp.float32),
    compiler_params=pltpu.CompilerParams(use_tc_tiling_on_sc=False),
    scratch_types=[
        pltpu.VMEM((_NPASS * _NCHUNK, _CHUNK), jnp.int32),
        pltpu.VMEM((_PASS, EMBED), jnp.float32),
        pltpu.SemaphoreType.DMA,
    ],
)
def _gather_sc(idx_hbm, tab_hbm, out_hbm, idx_v, rows_v, sem):
    base = _wid() * _B_PER_W
    pltpu.sync_copy(idx_hbm.at[_wid()], idx_v)
    for p in range(_NPASS):
        copies = []
        for j in range(_NCHUNK):
            dst = rows_v.at[pl.ds(j * _CHUNK, _CHUNK)]
            copies.append(
                pltpu.async_copy(tab_hbm.at[idx_v.at[p * _NCHUNK + j]], dst, sem))
        for c in copies:
            c.wait()
        pltpu.sync_copy(rows_v, out_hbm.at[pl.ds(base + p * _PASS, _PASS)])


@functools.partial(
    pl.kernel,
    mesh=_mesh,
    out_type=jax.ShapeDtypeStruct((BATCH, EMBED), jnp.float32),
    compiler_params=pltpu.CompilerParams(use_tc_tiling_on_sc=False),
    scratch_types=[
        pltpu.VMEM((_NPASS * _NCHUNK, _CHUNK), jnp.int32),
        pltpu.VMEM((_PASS, EMBED), jnp.float32),
        pltpu.VMEM((_PASS, EMBED), jnp.float32),
        pltpu.SemaphoreType.DMA,
    ],
)
def _gather_mul_sc(idx_hbm, tab_hbm, other_hbm, out_hbm, idx_v, rows_v, oth_v, sem):
    base = _wid() * _B_PER_W
    pltpu.sync_copy(idx_hbm.at[_wid()], idx_v)
    for p in range(_NPASS):
        copies = [pltpu.async_copy(
            other_hbm.at[pl.ds(base + p * _PASS, _PASS)], oth_v, sem)]
        for j in range(_NCHUNK):
            dst = rows_v.at[pl.ds(j * _CHUNK, _CHUNK)]
            copies.append(
                pltpu.async_copy(tab_hbm.at[idx_v.at[p * _NCHUNK + j]], dst, sem))
        for c in copies:
            c.wait()

        def row_body(r, _):
            for cbase in range(0, EMBED, LANES):
                sl = pl.ds(cbase, LANES)
                rows_v[r, sl] = rows_v[r, sl] * oth_v[r, sl]
            return 0

        lax.fori_loop(0, _PASS, row_body, 0)
        pltpu.sync_copy(rows_v, out_hbm.at[pl.ds(base + p * _PASS, _PASS)])


def kernel(user_indices, item_indices, user_table, item_table):
    uidx = user_indices.astype(jnp.int32).reshape(_NW, _NPASS * _NCHUNK, _CHUNK)
    iidx = item_indices.astype(jnp.int32).reshape(_NW, _NPASS * _NCHUNK, _CHUNK)
    user_rows = _gather_sc(uidx, user_table)
    return _gather_mul_sc(iidx, item_table, user_rows)
